# initial kernel scaffold (unmeasured)
import jax
import jax.numpy as jnp
from jax import lax
from jax.experimental import pallas as pl
from jax.experimental.pallas import tpu as pltpu

N_DEV = 32
M = 4096
N = 2048
CH = M // N_DEV


def kernel(x, w_mat):
    partial = jnp.dot(x, w_mat, preferred_element_type=jnp.float32)

    def body(
        p_ref,
        out_ref,
        comm_ref,
        send_sems,
        recv_sems,
        credit_sems,
        ag_send_sems,
        ag_recv_sems,
        ag_credit_sems,
    ):
        del p_ref
        my = lax.axis_index("i")
        left = lax.rem(my - 1 + N_DEV, N_DEV)
        right = lax.rem(my + 1, N_DEV)

        barrier_sem = pltpu.get_barrier_semaphore()
        for nbr in (left, right):
            pl.semaphore_signal(
                barrier_sem,
                inc=1,
                device_id=(nbr,),
                device_id_type=pl.DeviceIdType.MESH,
            )
        pl.semaphore_wait(barrier_sem, 2)

        def chunk(ref, c):
            return ref.at[pl.ds(c * CH, CH), :]

        for s in range(N_DEV - 1):
            slot = s % 2
            sc = lax.rem(my - s + N_DEV, N_DEV)
            rc = lax.rem(my - s - 1 + 2 * N_DEV, N_DEV)
            if s >= 2:
                pl.semaphore_wait(credit_sems.at[slot], 1)
            rdma = pltpu.make_async_remote_copy(
                src_ref=chunk(out_ref, sc),
                dst_ref=comm_ref.at[slot],
                send_sem=send_sems.at[slot],
                recv_sem=recv_sems.at[slot],
                device_id=(right,),
                device_id_type=pl.DeviceIdType.MESH,
            )
            rdma.start()
            rdma.wait_send()
            rdma.wait_recv()
            acc = out_ref[pl.ds(rc * CH, CH), :] + comm_ref[slot]
            if s == N_DEV - 2:
                acc = acc * jax.nn.sigmoid(acc)
            out_ref[pl.ds(rc * CH, CH), :] = acc
            pl.semaphore_signal(
                credit_sems.at[slot],
                inc=1,
                device_id=(left,),
                device_id_type=pl.DeviceIdType.MESH,
            )
        pl.semaphore_wait(credit_sems.at[0], 1)
        pl.semaphore_wait(credit_sems.at[1], 1)

        for s in range(N_DEV - 1):
            slot = s % 2
            sc = lax.rem(my + 1 - s + 2 * N_DEV, N_DEV)
            if s >= 2:
                pl.semaphore_wait(ag_credit_sems.at[slot], 1)
            rdma = pltpu.make_async_remote_copy(
                src_ref=chunk(out_ref, sc),
                dst_ref=chunk(out_ref, sc),
                send_sem=ag_send_sems.at[slot],
                recv_sem=ag_recv_sems.at[slot],
                device_id=(right,),
                device_id_type=pl.DeviceIdType.MESH,
            )
            rdma.start()
            rdma.wait_send()
            rdma.wait_recv()
            pl.semaphore_signal(
                ag_credit_sems.at[slot],
                inc=1,
                device_id=(left,),
                device_id_type=pl.DeviceIdType.MESH,
            )
        pl.semaphore_wait(ag_credit_sems.at[0], 1)
        pl.semaphore_wait(ag_credit_sems.at[1], 1)

    return pl.pallas_call(
        body,
        out_shape=jax.ShapeDtypeStruct((M, N), jnp.float32),
        in_specs=[pl.BlockSpec(memory_space=pltpu.VMEM)],
        out_specs=pl.BlockSpec(memory_space=pltpu.VMEM),
        scratch_shapes=[
            pltpu.VMEM((2, CH, N), jnp.float32),
            pltpu.SemaphoreType.DMA((2,)),
            pltpu.SemaphoreType.DMA((2,)),
            pltpu.SemaphoreType.REGULAR((2,)),
            pltpu.SemaphoreType.DMA((2,)),
            pltpu.SemaphoreType.DMA((2,)),
            pltpu.SemaphoreType.REGULAR((2,)),
        ],
        input_output_aliases={0: 0},
        compiler_params=pltpu.CompilerParams(collective_id=0),
    )(partial)


# baseline (device time: 872012 ns/iter reference)
import jax
import jax.numpy as jnp
from jax import lax
from jax.experimental import pallas as pl
from jax.experimental.pallas import tpu as pltpu

N_DEV = 32
M = 4096
N = 2048
CH = M // N_DEV


def kernel(x, w_mat):
    partial = jnp.dot(x, w_mat, preferred_element_type=jnp.float32)

    def body(
        p_ref,
        out_ref,
        comm_ref,
        stage_ref,
        local_sems,
        send_sems,
        recv_sems,
        credit_sems,
        ag_send_sems,
        ag_recv_sems,
        ag_credit_sems,
    ):
        my = lax.axis_index("i")
        left = lax.rem(my - 1 + N_DEV, N_DEV)
        right = lax.rem(my + 1, N_DEV)

        barrier_sem = pltpu.get_barrier_semaphore()
        for nbr in (left, right):
            pl.semaphore_signal(
                barrier_sem,
                inc=1,
                device_id=(nbr,),
                device_id_type=pl.DeviceIdType.MESH,
            )
        pl.semaphore_wait(barrier_sem, 2)

        def chunk(ref, c):
            return ref.at[pl.ds(c * CH, CH), :]

        seed = pltpu.make_async_copy(
            chunk(p_ref, my), chunk(out_ref, my), local_sems.at[0]
        )
        seed.start()
        seed.wait()

        for s in range(N_DEV - 1):
            slot = s % 2
            sc = lax.rem(my - s + N_DEV, N_DEV)
            rc = lax.rem(my - s - 1 + 2 * N_DEV, N_DEV)
            stage = pltpu.make_async_copy(
                chunk(p_ref, rc), stage_ref.at[slot], local_sems.at[slot]
            )
            stage.start()
            if s >= 2:
                pl.semaphore_wait(credit_sems.at[slot], 1)
            rdma = pltpu.make_async_remote_copy(
                src_ref=chunk(out_ref, sc),
                dst_ref=comm_ref.at[slot],
                send_sem=send_sems.at[slot],
                recv_sem=recv_sems.at[slot],
                device_id=(right,),
                device_id_type=pl.DeviceIdType.MESH,
            )
            rdma.start()
            rdma.wait_send()
            rdma.wait_recv()
            stage.wait()
            acc = stage_ref[slot] + comm_ref[slot]
            if s == N_DEV - 2:
                acc = acc * jax.nn.sigmoid(acc)
            out_ref[pl.ds(rc * CH, CH), :] = acc
            pl.semaphore_signal(
                credit_sems.at[slot],
                inc=1,
                device_id=(left,),
                device_id_type=pl.DeviceIdType.MESH,
            )
        pl.semaphore_wait(credit_sems.at[0], 1)
        pl.semaphore_wait(credit_sems.at[1], 1)

        for s in range(N_DEV - 1):
            slot = s % 2
            sc = lax.rem(my + 1 - s + 2 * N_DEV, N_DEV)
            if s >= 2:
                pl.semaphore_wait(ag_credit_sems.at[slot], 1)
            rdma = pltpu.make_async_remote_copy(
                src_ref=chunk(out_ref, sc),
                dst_ref=chunk(out_ref, sc),
                send_sem=ag_send_sems.at[slot],
                recv_sem=ag_recv_sems.at[slot],
                device_id=(right,),
                device_id_type=pl.DeviceIdType.MESH,
            )
            rdma.start()
            rdma.wait_send()
            rdma.wait_recv()
            pl.semaphore_signal(
                ag_credit_sems.at[slot],
                inc=1,
                device_id=(left,),
                device_id_type=pl.DeviceIdType.MESH,
            )
        pl.semaphore_wait(ag_credit_sems.at[0], 1)
        pl.semaphore_wait(ag_credit_sems.at[1], 1)

    return pl.pallas_call(
        body,
        out_shape=jax.ShapeDtypeStruct((M, N), jnp.float32),
        in_specs=[pl.BlockSpec(memory_space=pltpu.MemorySpace.HBM)],
        out_specs=pl.BlockSpec(memory_space=pltpu.VMEM),
        scratch_shapes=[
            pltpu.VMEM((2, CH, N), jnp.float32),
            pltpu.VMEM((2, CH, N), jnp.float32),
            pltpu.SemaphoreType.DMA((2,)),
            pltpu.SemaphoreType.DMA((2,)),
            pltpu.SemaphoreType.DMA((2,)),
            pltpu.SemaphoreType.REGULAR((2,)),
            pltpu.SemaphoreType.DMA((2,)),
            pltpu.SemaphoreType.DMA((2,)),
            pltpu.SemaphoreType.REGULAR((2,)),
        ],
        input_output_aliases={0: 0},
        compiler_params=pltpu.CompilerParams(
            collective_id=0, vmem_limit_bytes=48 * 1024 * 1024
        ),
    )(partial)
